# baseline (speedup 1.0000x reference)
import functools

import jax
import jax.numpy as jnp
from jax import lax
from jax.experimental import pallas as pl
from jax.experimental.pallas import tpu as pltpu
from jax.experimental.pallas import tpu_sc as plsc

_NC = 2
_NS = 16
_NW = _NC * _NS
_BATCH = 64
_NBUF = 4
_DEGW = 128
_NORMW = 16
_RB = 512


def _build_deg_kernel(npad, nb, rows_pt):
    mesh = plsc.VectorSubcoreMesh(core_axis_name="c", subcore_axis_name="s")

    @functools.partial(
        pl.kernel,
        mesh=mesh,
        out_type=jax.ShapeDtypeStruct((_NC, npad, _DEGW), jnp.float32),
        scratch_types=[
            pltpu.VMEM((nb, _BATCH), jnp.int32),
            pltpu.VMEM((_BATCH, _DEGW), jnp.float32),
            pltpu.VMEM_SHARED((npad, _DEGW), jnp.float32),
        ],
    )
    def deg_kernel(dst_hbm, ones_hbm, zeros_hbm, out_hbm, dst_v, ones_v, acc):
        c = lax.axis_index("c")
        s = lax.axis_index("s")
        wid = c * _NS + s
        r0 = s * rows_pt
        pltpu.sync_copy(zeros_hbm.at[pl.ds(r0, rows_pt)], acc.at[pl.ds(r0, rows_pt)])
        pltpu.sync_copy(ones_hbm, ones_v)
        pltpu.sync_copy(dst_hbm.at[wid], dst_v)
        plsc.subcore_barrier()

        def body(b, carry):
            pltpu.sync_copy(ones_v, acc.at[dst_v.at[b]], add=True)
            return carry

        lax.fori_loop(0, nb, body, 0)
        plsc.subcore_barrier()
        pltpu.sync_copy(acc.at[pl.ds(r0, rows_pt)], out_hbm.at[c, pl.ds(r0, rows_pt)])

    return deg_kernel


def _build_agg_kernel(npad, d, nb, rows_pt):
    mesh = plsc.VectorSubcoreMesh(core_axis_name="c", subcore_axis_name="s")

    nchunk = 4
    cb = nb // nchunk
    assert nb % nchunk == 0 and cb % 8 == 0 and cb % _NBUF == 0

    @functools.partial(
        pl.kernel,
        mesh=mesh,
        out_type=jax.ShapeDtypeStruct((_NC, npad, d), jnp.float32),
        scratch_types=[
            pltpu.VMEM((cb, _BATCH), jnp.int32),
            pltpu.VMEM((cb, _BATCH), jnp.int32),
            [pltpu.VMEM((_BATCH, d), jnp.float32) for _ in range(_NBUF)],
            pltpu.VMEM_SHARED((npad, d), jnp.float32),
            [pltpu.SemaphoreType.DMA for _ in range(_NBUF)],
        ],
    )
    def agg_kernel(src_hbm, dst_hbm, h_hbm, zeros_hbm, out_hbm,
                   src_v, dst_v, rows, acc, sems):
        c = lax.axis_index("c")
        s = lax.axis_index("s")
        wid = c * _NS + s
        r0 = s * rows_pt
        pltpu.sync_copy(zeros_hbm.at[pl.ds(r0, rows_pt)], acc.at[pl.ds(r0, rows_pt)])
        plsc.subcore_barrier()

        for h in range(nchunk):
            pltpu.sync_copy(src_hbm.at[wid, pl.ds(h * cb, cb)], src_v)
            pltpu.sync_copy(dst_hbm.at[wid, pl.ds(h * cb, cb)], dst_v)
            for j in range(_NBUF):
                pltpu.async_copy(h_hbm.at[src_v.at[j]], rows[j], sems[j])

            def body(i, carry):
                for j in range(_NBUF):
                    b = _NBUF * i + j
                    pltpu.make_async_copy(
                        h_hbm.at[src_v.at[b]], rows[j], sems[j]).wait()
                    pltpu.sync_copy(rows[j], acc.at[dst_v.at[b]], add=True)

                    @pl.when(b + _NBUF < cb)
                    def _():
                        pltpu.async_copy(
                            h_hbm.at[src_v.at[b + _NBUF]], rows[j], sems[j])

                return carry

            lax.fori_loop(0, cb // _NBUF, body, 0)

        plsc.subcore_barrier()
        pltpu.sync_copy(acc.at[pl.ds(r0, rows_pt)], out_hbm.at[c, pl.ds(r0, rows_pt)])

    return agg_kernel


def _norm_from_deg(deg_ref):
    d = deg_ref[0, :, 0:1] + deg_ref[1, :, 0:1]
    return jnp.where(d > 0.0, lax.rsqrt(d), 0.0)


def _tc1_body(deg_ref, feat_ref, lam_ref, h1_ref, n16_ref):
    norm = _norm_from_deg(deg_ref)
    h1_ref[...] = feat_ref[...] * (norm * lam_ref[0, 0])
    n16_ref[...] = jnp.broadcast_to(norm, (norm.shape[0], _NORMW))


def _tc2_body(n16_ref, s1_ref, feat_ref, lam_ref, g1_ref, h2_ref):
    norm = n16_ref[:, 0:1]
    g1 = (s1_ref[0] + s1_ref[1]) * norm
    g1_ref[...] = g1
    h2_ref[...] = (g1 - feat_ref[...]) * (norm * lam_ref[0, 0])


def _tc3_body(n16_ref, feat_ref, g1_ref, s2_ref, a_ref, b_ref, c_ref,
              bias_ref, out_ref):
    norm = n16_ref[:, 0:1]
    t = (s2_ref[0] + s2_ref[1]) * norm
    acc = jnp.dot(feat_ref[...], a_ref[...], preferred_element_type=jnp.float32)
    acc = acc + jnp.dot(g1_ref[...], b_ref[...], preferred_element_type=jnp.float32)
    acc = acc + jnp.dot(t, c_ref[...], preferred_element_type=jnp.float32)
    out_ref[...] = acc + bias_ref[0:1, :]


def _deg_spec():
    return pl.BlockSpec((_NC, _RB, _DEGW), lambda i: (0, i, 0))


def _row_spec(d):
    return pl.BlockSpec((_RB, d), lambda i: (i, 0))


def _part_spec(d):
    return pl.BlockSpec((_NC, _RB, d), lambda i: (0, i, 0))


def _full_spec(shape):
    return pl.BlockSpec(shape, lambda i: tuple(0 for _ in shape))


def kernel(feat, edge_index, lambda_max, W0, W1, W2, bias):
    n, d = feat.shape
    e = edge_index.shape[1]

    npad = -(-(n + 8) // (_NW * 8)) * (_NW * 8)
    rows_pt = npad // _NS
    nb = -(-e // (_NW * _BATCH))
    nb = -(-nb // 32) * 32
    ept = nb * _BATCH
    epad = _NW * ept

    src = edge_index[0]
    dst = edge_index[1]
    srcp = jnp.concatenate(
        [src, jnp.zeros((epad - e,), jnp.int32)]).reshape(_NW, nb, _BATCH)
    pad_span = max(1, min(_BATCH, npad - n))
    pad_dst = n + (jnp.arange(epad - e, dtype=jnp.int32) % pad_span)
    dstp = jnp.concatenate([dst, pad_dst]).reshape(_NW, nb, _BATCH)
    featp = jnp.zeros((npad, d), jnp.float32).at[:n, :].set(feat)

    zeros_deg = jnp.zeros((npad, _DEGW), jnp.float32)
    zeros_row = jnp.zeros((npad, d), jnp.float32)
    ones_deg = jnp.ones((_BATCH, _DEGW), jnp.float32)
    lam = jnp.full((8, 128), 2.0 / lambda_max, jnp.float32)

    deg_k = _build_deg_kernel(npad, nb, rows_pt)
    agg_k = _build_agg_kernel(npad, d, nb, rows_pt)

    degp = deg_k(dstp, ones_deg, zeros_deg)

    grid = npad // _RB
    h1, norm16 = pl.pallas_call(
        _tc1_body,
        grid=(grid,),
        in_specs=[_deg_spec(), _row_spec(d), _full_spec((8, 128))],
        out_specs=[_row_spec(d), _row_spec(_NORMW)],
        out_shape=[jax.ShapeDtypeStruct((npad, d), jnp.float32),
                   jax.ShapeDtypeStruct((npad, _NORMW), jnp.float32)],
    )(degp, featp, lam)

    s1 = agg_k(srcp, dstp, h1, zeros_row)

    g1, h2 = pl.pallas_call(
        _tc2_body,
        grid=(grid,),
        in_specs=[_row_spec(_NORMW), _part_spec(d), _row_spec(d),
                  _full_spec((8, 128))],
        out_specs=[_row_spec(d), _row_spec(d)],
        out_shape=[jax.ShapeDtypeStruct((npad, d), jnp.float32),
                   jax.ShapeDtypeStruct((npad, d), jnp.float32)],
    )(norm16, s1, featp, lam)

    s2 = agg_k(srcp, dstp, h2, zeros_row)

    wa = W0 - W1 + W2
    wb = W1 - 2.0 * W2
    wc = 2.0 * W2
    bias2 = jnp.broadcast_to(bias[None, :], (8, d))

    out = pl.pallas_call(
        _tc3_body,
        grid=(grid,),
        in_specs=[_row_spec(_NORMW), _row_spec(d), _row_spec(d), _part_spec(d),
                  _full_spec((d, d)), _full_spec((d, d)), _full_spec((d, d)),
                  _full_spec((8, d))],
        out_specs=_row_spec(d),
        out_shape=jax.ShapeDtypeStruct((npad, d), jnp.float32),
    )(norm16, featp, g1, s2, wa, wb, wc, bias2)

    return out[:n]


# speedup vs baseline: 1.1231x; 1.1231x over previous
import functools

import jax
import jax.numpy as jnp
from jax import lax
from jax.experimental import pallas as pl
from jax.experimental.pallas import tpu as pltpu
from jax.experimental.pallas import tpu_sc as plsc

_NC = 2
_NS = 16
_NW = _NC * _NS
_BATCH = 128
_NBUF = 2
_DEGW = 128
_NORMW = 16
_RB = 512


def _build_deg_kernel(npad, nb, rows_pt):
    mesh = plsc.VectorSubcoreMesh(core_axis_name="c", subcore_axis_name="s")

    @functools.partial(
        pl.kernel,
        mesh=mesh,
        out_type=jax.ShapeDtypeStruct((_NC, npad, _DEGW), jnp.float32),
        scratch_types=[
            pltpu.VMEM((nb, _BATCH), jnp.int32),
            pltpu.VMEM((_BATCH, _DEGW), jnp.float32),
            pltpu.VMEM_SHARED((npad, _DEGW), jnp.float32),
            pltpu.SemaphoreType.DMA,
        ],
    )
    def deg_kernel(dst_hbm, ones_hbm, zeros_hbm, out_hbm, dst_v, ones_v, acc, sem):
        c = lax.axis_index("c")
        s = lax.axis_index("s")
        wid = c * _NS + s
        r0 = s * rows_pt
        pltpu.sync_copy(zeros_hbm.at[pl.ds(r0, rows_pt)], acc.at[pl.ds(r0, rows_pt)])
        pltpu.sync_copy(ones_hbm, ones_v)
        pltpu.sync_copy(dst_hbm.at[wid], dst_v)
        plsc.subcore_barrier()

        win = 4

        def body(b, carry):
            pltpu.async_copy(ones_v, acc.at[dst_v.at[b]], sem, add=True)

            @pl.when(b >= win)
            def _():
                pltpu.make_async_copy(ones_v, acc.at[dst_v.at[0]], sem).wait()

            return carry

        lax.fori_loop(0, nb, body, 0)

        def drain(b, carry):
            pltpu.make_async_copy(ones_v, acc.at[dst_v.at[0]], sem).wait()
            return carry

        lax.fori_loop(0, min(win, nb), drain, 0)
        plsc.subcore_barrier()
        pltpu.sync_copy(acc.at[pl.ds(r0, rows_pt)], out_hbm.at[c, pl.ds(r0, rows_pt)])

    return deg_kernel


def _build_agg_kernel(npad, d, nb, rows_pt):
    mesh = plsc.VectorSubcoreMesh(core_axis_name="c", subcore_axis_name="s")

    nchunk = 2
    cb = nb // nchunk
    assert nb % nchunk == 0 and cb % 8 == 0 and cb % _NBUF == 0

    @functools.partial(
        pl.kernel,
        mesh=mesh,
        out_type=jax.ShapeDtypeStruct((_NC, npad, d), jnp.float32),
        scratch_types=[
            pltpu.VMEM((cb, _BATCH), jnp.int32),
            pltpu.VMEM((cb, _BATCH), jnp.int32),
            [pltpu.VMEM((_BATCH, d), jnp.float32) for _ in range(_NBUF)],
            pltpu.VMEM_SHARED((npad, d), jnp.float32),
            [pltpu.SemaphoreType.DMA for _ in range(_NBUF)],
        ],
    )
    def agg_kernel(src_hbm, dst_hbm, h_hbm, zeros_hbm, out_hbm,
                   src_v, dst_v, rows, acc, sems):
        c = lax.axis_index("c")
        s = lax.axis_index("s")
        wid = c * _NS + s
        r0 = s * rows_pt
        pltpu.sync_copy(zeros_hbm.at[pl.ds(r0, rows_pt)], acc.at[pl.ds(r0, rows_pt)])
        plsc.subcore_barrier()

        for h in range(nchunk):
            pltpu.sync_copy(src_hbm.at[wid, pl.ds(h * cb, cb)], src_v)
            pltpu.sync_copy(dst_hbm.at[wid, pl.ds(h * cb, cb)], dst_v)
            for j in range(_NBUF):
                pltpu.async_copy(h_hbm.at[src_v.at[j]], rows[j], sems[j])

            def body(i, carry):
                for j in range(_NBUF):
                    b = _NBUF * i + j
                    pltpu.make_async_copy(
                        h_hbm.at[src_v.at[b]], rows[j], sems[j]).wait()
                    pltpu.sync_copy(rows[j], acc.at[dst_v.at[b]], add=True)

                    @pl.when(b + _NBUF < cb)
                    def _():
                        pltpu.async_copy(
                            h_hbm.at[src_v.at[b + _NBUF]], rows[j], sems[j])

                return carry

            lax.fori_loop(0, cb // _NBUF, body, 0)

        plsc.subcore_barrier()
        pltpu.sync_copy(acc.at[pl.ds(r0, rows_pt)], out_hbm.at[c, pl.ds(r0, rows_pt)])

    return agg_kernel


def _norm_from_deg(deg_ref):
    d = deg_ref[0, :, 0:1] + deg_ref[1, :, 0:1]
    return jnp.where(d > 0.0, lax.rsqrt(d), 0.0)


def _tc1_body(deg_ref, feat_ref, lam_ref, h1_ref, n16_ref):
    norm = _norm_from_deg(deg_ref)
    h1_ref[...] = feat_ref[...] * (norm * lam_ref[0, 0])
    n16_ref[...] = jnp.broadcast_to(norm, (norm.shape[0], _NORMW))


def _tc2_body(n16_ref, s1_ref, feat_ref, lam_ref, g1_ref, h2_ref):
    norm = n16_ref[:, 0:1]
    g1 = (s1_ref[0] + s1_ref[1]) * norm
    g1_ref[...] = g1
    h2_ref[...] = (g1 - feat_ref[...]) * (norm * lam_ref[0, 0])


def _tc3_body(n16_ref, feat_ref, g1_ref, s2_ref, a_ref, b_ref, c_ref,
              bias_ref, out_ref):
    norm = n16_ref[:, 0:1]
    t = (s2_ref[0] + s2_ref[1]) * norm
    acc = jnp.dot(feat_ref[...], a_ref[...], preferred_element_type=jnp.float32)
    acc = acc + jnp.dot(g1_ref[...], b_ref[...], preferred_element_type=jnp.float32)
    acc = acc + jnp.dot(t, c_ref[...], preferred_element_type=jnp.float32)
    out_ref[...] = acc + bias_ref[0:1, :]


def _deg_spec():
    return pl.BlockSpec((_NC, _RB, _DEGW), lambda i: (0, i, 0))


def _row_spec(d):
    return pl.BlockSpec((_RB, d), lambda i: (i, 0))


def _part_spec(d):
    return pl.BlockSpec((_NC, _RB, d), lambda i: (0, i, 0))


def _full_spec(shape):
    return pl.BlockSpec(shape, lambda i: tuple(0 for _ in shape))


def kernel(feat, edge_index, lambda_max, W0, W1, W2, bias):
    n, d = feat.shape
    e = edge_index.shape[1]

    npad = -(-(n + 8) // (_NW * 8)) * (_NW * 8)
    rows_pt = npad // _NS
    nb = -(-e // (_NW * _BATCH))
    nb = -(-nb // 16) * 16
    ept = nb * _BATCH
    epad = _NW * ept

    src = edge_index[0]
    dst = edge_index[1]
    srcp = jnp.concatenate(
        [src, jnp.zeros((epad - e,), jnp.int32)]).reshape(_NW, nb, _BATCH)
    pad_span = max(1, min(_BATCH, npad - n))
    pad_dst = n + (jnp.arange(epad - e, dtype=jnp.int32) % pad_span)
    dstp = jnp.concatenate([dst, pad_dst]).reshape(_NW, nb, _BATCH)
    featp = jnp.zeros((npad, d), jnp.float32).at[:n, :].set(feat)

    zeros_deg = jnp.zeros((npad, _DEGW), jnp.float32)
    zeros_row = jnp.zeros((npad, d), jnp.float32)
    ones_deg = jnp.ones((_BATCH, _DEGW), jnp.float32)
    lam = jnp.full((8, 128), 2.0 / lambda_max, jnp.float32)

    deg_k = _build_deg_kernel(npad, nb, rows_pt)
    agg_k = _build_agg_kernel(npad, d, nb, rows_pt)

    degp = deg_k(dstp, ones_deg, zeros_deg)

    grid = npad // _RB
    h1, norm16 = pl.pallas_call(
        _tc1_body,
        grid=(grid,),
        in_specs=[_deg_spec(), _row_spec(d), _full_spec((8, 128))],
        out_specs=[_row_spec(d), _row_spec(_NORMW)],
        out_shape=[jax.ShapeDtypeStruct((npad, d), jnp.float32),
                   jax.ShapeDtypeStruct((npad, _NORMW), jnp.float32)],
    )(degp, featp, lam)

    s1 = agg_k(srcp, dstp, h1, zeros_row)

    g1, h2 = pl.pallas_call(
        _tc2_body,
        grid=(grid,),
        in_specs=[_row_spec(_NORMW), _part_spec(d), _row_spec(d),
                  _full_spec((8, 128))],
        out_specs=[_row_spec(d), _row_spec(d)],
        out_shape=[jax.ShapeDtypeStruct((npad, d), jnp.float32),
                   jax.ShapeDtypeStruct((npad, d), jnp.float32)],
    )(norm16, s1, featp, lam)

    s2 = agg_k(srcp, dstp, h2, zeros_row)

    wa = W0 - W1 + W2
    wb = W1 - 2.0 * W2
    wc = 2.0 * W2
    bias2 = jnp.broadcast_to(bias[None, :], (8, d))

    out = pl.pallas_call(
        _tc3_body,
        grid=(grid,),
        in_specs=[_row_spec(_NORMW), _row_spec(d), _row_spec(d), _part_spec(d),
                  _full_spec((d, d)), _full_spec((d, d)), _full_spec((d, d)),
                  _full_spec((8, d))],
        out_specs=_row_spec(d),
        out_shape=jax.ShapeDtypeStruct((npad, d), jnp.float32),
    )(norm16, featp, g1, s2, wa, wb, wc, bias2)

    return out[:n]


# speedup vs baseline: 3.0535x; 2.7188x over previous
import functools

import jax
import jax.numpy as jnp
from jax import lax
from jax.experimental import pallas as pl
from jax.experimental.pallas import tpu as pltpu
from jax.experimental.pallas import tpu_sc as plsc

_NC = 2
_NS = 16
_NW = _NC * _NS
_BATCH = 128
_NBUF = 2
_DEGW = 128
_NORMW = 16
_RB = 512


def _build_deg_kernel(npad, nb, rows_pt):
    mesh = plsc.VectorSubcoreMesh(core_axis_name="c", subcore_axis_name="s")

    @functools.partial(
        pl.kernel,
        mesh=mesh,
        out_type=jax.ShapeDtypeStruct((_NC, npad, _DEGW), jnp.float32),
        scratch_types=[
            pltpu.VMEM((nb, _BATCH), jnp.int32),
            pltpu.VMEM((_BATCH, _DEGW), jnp.float32),
            pltpu.VMEM_SHARED((npad, _DEGW), jnp.float32),
            pltpu.SemaphoreType.DMA,
        ],
    )
    def deg_kernel(dst_hbm, ones_hbm, zeros_hbm, out_hbm, dst_v, ones_v, acc, sem):
        c = lax.axis_index("c")
        s = lax.axis_index("s")
        wid = c * _NS + s
        r0 = s * rows_pt
        pltpu.sync_copy(zeros_hbm.at[pl.ds(r0, rows_pt)], acc.at[pl.ds(r0, rows_pt)])
        pltpu.sync_copy(ones_hbm, ones_v)
        pltpu.sync_copy(dst_hbm.at[wid], dst_v)
        plsc.subcore_barrier()

        win = 4

        def body(b, carry):
            pltpu.async_copy(ones_v, acc.at[dst_v.at[b]], sem, add=True)

            @pl.when(b >= win)
            def _():
                pltpu.make_async_copy(ones_v, acc.at[dst_v.at[0]], sem).wait()

            return carry

        lax.fori_loop(0, nb, body, 0)

        def drain(b, carry):
            pltpu.make_async_copy(ones_v, acc.at[dst_v.at[0]], sem).wait()
            return carry

        lax.fori_loop(0, min(win, nb), drain, 0)
        plsc.subcore_barrier()
        pltpu.sync_copy(acc.at[pl.ds(r0, rows_pt)], out_hbm.at[c, pl.ds(r0, rows_pt)])

    return deg_kernel


def _build_agg_kernel(npad, d, nb, rows_pt):
    mesh = plsc.VectorSubcoreMesh(core_axis_name="c", subcore_axis_name="s")

    nchunk = 2
    cb = nb // nchunk
    assert nb % nchunk == 0 and cb % 8 == 0 and cb % _NBUF == 0

    @functools.partial(
        pl.kernel,
        mesh=mesh,
        out_type=jax.ShapeDtypeStruct((_NC, npad, d), jnp.float32),
        scratch_types=[
            pltpu.VMEM((cb, _BATCH), jnp.int32),
            pltpu.VMEM((cb, _BATCH), jnp.int32),
            [pltpu.VMEM((_BATCH, d), jnp.float32) for _ in range(_NBUF)],
            pltpu.VMEM_SHARED((npad, d), jnp.float32),
            [pltpu.SemaphoreType.DMA for _ in range(_NBUF)],
        ],
    )
    def agg_kernel(src_hbm, dst_hbm, h_hbm, zeros_hbm, out_hbm,
                   src_v, dst_v, rows, acc, sems):
        c = lax.axis_index("c")
        s = lax.axis_index("s")
        wid = c * _NS + s
        r0 = s * rows_pt
        pltpu.sync_copy(zeros_hbm.at[pl.ds(r0, rows_pt)], acc.at[pl.ds(r0, rows_pt)])
        plsc.subcore_barrier()

        for h in range(nchunk):
            pltpu.sync_copy(src_hbm.at[wid, pl.ds(h * cb, cb)], src_v)
            pltpu.sync_copy(dst_hbm.at[wid, pl.ds(h * cb, cb)], dst_v)
            for j in range(_NBUF):
                pltpu.async_copy(h_hbm.at[src_v.at[j]], rows[j], sems[j])

            def body(i, carry):
                for j in range(_NBUF):
                    b = _NBUF * i + j
                    pltpu.make_async_copy(
                        h_hbm.at[src_v.at[b]], rows[j], sems[j]).wait()
                    pltpu.sync_copy(rows[j], acc.at[dst_v.at[b]], add=True)

                    @pl.when(b + _NBUF < cb)
                    def _():
                        pltpu.async_copy(
                            h_hbm.at[src_v.at[b + _NBUF]], rows[j], sems[j])

                return carry

            lax.fori_loop(0, cb // _NBUF, body, 0)

        plsc.subcore_barrier()
        pltpu.sync_copy(acc.at[pl.ds(r0, rows_pt)], out_hbm.at[c, pl.ds(r0, rows_pt)])

    return agg_kernel


def _norm_from_deg(deg_ref):
    d = deg_ref[0, :, 0:1] + deg_ref[1, :, 0:1]
    return jnp.where(d > 0.0, lax.rsqrt(d), 0.0)


def _tc1_body(deg_ref, feat_ref, lam_ref, h1_ref, n16_ref):
    norm = _norm_from_deg(deg_ref)
    h1_ref[...] = feat_ref[...] * (norm * lam_ref[0, 0])
    n16_ref[...] = jnp.broadcast_to(norm, (norm.shape[0], _NORMW))


def _tc2_body(n16_ref, s1_ref, feat_ref, lam_ref, g1_ref, h2_ref):
    norm = n16_ref[:, 0:1]
    g1 = (s1_ref[0] + s1_ref[1]) * norm
    g1_ref[...] = g1
    h2_ref[...] = (g1 - feat_ref[...]) * (norm * lam_ref[0, 0])


def _tc3_body(n16_ref, feat_ref, g1_ref, s2_ref, a_ref, b_ref, c_ref,
              bias_ref, out_ref):
    norm = n16_ref[:, 0:1]
    t = (s2_ref[0] + s2_ref[1]) * norm
    acc = jnp.dot(feat_ref[...], a_ref[...], preferred_element_type=jnp.float32)
    acc = acc + jnp.dot(g1_ref[...], b_ref[...], preferred_element_type=jnp.float32)
    acc = acc + jnp.dot(t, c_ref[...], preferred_element_type=jnp.float32)
    out_ref[...] = acc + bias_ref[0:1, :]


def _deg_spec():
    return pl.BlockSpec((_NC, _RB, _DEGW), lambda i: (0, i, 0))


def _row_spec(d):
    return pl.BlockSpec((_RB, d), lambda i: (i, 0))


def _part_spec(d):
    return pl.BlockSpec((_NC, _RB, d), lambda i: (0, i, 0))


def _full_spec(shape):
    return pl.BlockSpec(shape, lambda i: tuple(0 for _ in shape))


def kernel(feat, edge_index, lambda_max, W0, W1, W2, bias):
    n, d = feat.shape
    e = edge_index.shape[1]

    npad = -(-(n + 8) // (_NW * 8)) * (_NW * 8)
    rows_pt = npad // _NS
    nb = -(-e // (_NW * _BATCH))
    nb = -(-nb // 16) * 16
    ept = nb * _BATCH
    epad = _NW * ept

    src = edge_index[0]
    dst = edge_index[1]
    pad_src = jnp.arange(epad - e, dtype=jnp.int32) % min(n, _BATCH)
    srcp = jnp.concatenate([src, pad_src]).reshape(_NW, nb, _BATCH)
    pad_span = max(1, min(_BATCH, npad - n))
    pad_dst = n + (jnp.arange(epad - e, dtype=jnp.int32) % pad_span)
    dstp = jnp.concatenate([dst, pad_dst]).reshape(_NW, nb, _BATCH)
    featp = jnp.zeros((npad, d), jnp.float32).at[:n, :].set(feat)

    zeros_deg = jnp.zeros((npad, _DEGW), jnp.float32)
    zeros_row = jnp.zeros((npad, d), jnp.float32)
    ones_deg = jnp.ones((_BATCH, _DEGW), jnp.float32)
    lam = jnp.full((8, 128), 2.0 / lambda_max, jnp.float32)

    deg_k = _build_deg_kernel(npad, nb, rows_pt)
    agg_k = _build_agg_kernel(npad, d, nb, rows_pt)

    degp = deg_k(dstp, ones_deg, zeros_deg)

    grid = npad // _RB
    h1, norm16 = pl.pallas_call(
        _tc1_body,
        grid=(grid,),
        in_specs=[_deg_spec(), _row_spec(d), _full_spec((8, 128))],
        out_specs=[_row_spec(d), _row_spec(_NORMW)],
        out_shape=[jax.ShapeDtypeStruct((npad, d), jnp.float32),
                   jax.ShapeDtypeStruct((npad, _NORMW), jnp.float32)],
    )(degp, featp, lam)

    s1 = agg_k(srcp, dstp, h1, zeros_row)

    g1, h2 = pl.pallas_call(
        _tc2_body,
        grid=(grid,),
        in_specs=[_row_spec(_NORMW), _part_spec(d), _row_spec(d),
                  _full_spec((8, 128))],
        out_specs=[_row_spec(d), _row_spec(d)],
        out_shape=[jax.ShapeDtypeStruct((npad, d), jnp.float32),
                   jax.ShapeDtypeStruct((npad, d), jnp.float32)],
    )(norm16, s1, featp, lam)

    s2 = agg_k(srcp, dstp, h2, zeros_row)

    wa = W0 - W1 + W2
    wb = W1 - 2.0 * W2
    wc = 2.0 * W2
    bias2 = jnp.broadcast_to(bias[None, :], (8, d))

    out = pl.pallas_call(
        _tc3_body,
        grid=(grid,),
        in_specs=[_row_spec(_NORMW), _row_spec(d), _row_spec(d), _part_spec(d),
                  _full_spec((d, d)), _full_spec((d, d)), _full_spec((d, d)),
                  _full_spec((8, d))],
        out_specs=_row_spec(d),
        out_shape=jax.ShapeDtypeStruct((npad, d), jnp.float32),
    )(norm16, featp, g1, s2, wa, wb, wc, bias2)

    return out[:n]
